# Initial kernel scaffold; baseline (speedup 1.0000x reference)
import functools

import jax
import jax.numpy as jnp
from jax import lax
from jax.experimental import pallas as pl
from jax.experimental.pallas import tpu as pltpu
from jax.experimental.pallas import tpu_sc as plsc

NC = 2
NS = 16
CHUNK = 80
WD = 8


def _make_deg(N, E):
    NW = NC * NS
    epw = E // NW
    nch = epw // CHUNK
    rpt = N // NS
    mesh = plsc.VectorSubcoreMesh(core_axis_name="c", subcore_axis_name="s")

    @functools.partial(
        pl.kernel,
        out_type=jax.ShapeDtypeStruct((NC, N, WD), jnp.float32),
        mesh=mesh,
        scratch_types=[
            pltpu.VMEM_SHARED((N, WD), jnp.float32),
            pltpu.VMEM((CHUNK,), jnp.int32),
            pltpu.VMEM((CHUNK, WD), jnp.float32),
        ],
    )
    def deg(dst_hbm, ones_hbm, zeros_hbm, out_hbm, deg_sh, idx_v, ones_v):
        cid = lax.axis_index("c")
        sid = lax.axis_index("s")
        wid = cid * NS + sid
        row0 = sid * rpt
        pltpu.sync_copy(zeros_hbm.at[pl.ds(row0, rpt)], deg_sh.at[pl.ds(row0, rpt)])
        pltpu.sync_copy(ones_hbm, ones_v)
        plsc.subcore_barrier()
        ebase = wid * epw

        def body(i, carry):
            off = pl.multiple_of(ebase + i * CHUNK, 8)
            pltpu.sync_copy(dst_hbm.at[pl.ds(off, CHUNK)], idx_v)
            pltpu.sync_copy(ones_v, deg_sh.at[idx_v], add=True)
            return carry

        lax.fori_loop(0, nch, body, 0)
        plsc.subcore_barrier()
        pltpu.sync_copy(deg_sh.at[pl.ds(row0, rpt)], out_hbm.at[cid, pl.ds(row0, rpt)])

    return deg


def _make_agg(N, D, E):
    NW = NC * NS
    epw = E // NW
    nch = epw // CHUNK
    rpt = N // NS
    mesh = plsc.VectorSubcoreMesh(core_axis_name="c", subcore_axis_name="s")

    @functools.partial(
        pl.kernel,
        out_type=jax.ShapeDtypeStruct((NC, N, D), jnp.float32),
        mesh=mesh,
        scratch_types=[
            pltpu.VMEM_SHARED((N, D), jnp.float32),
            pltpu.VMEM((CHUNK,), jnp.int32),
            pltpu.VMEM((CHUNK,), jnp.int32),
            pltpu.VMEM((CHUNK, D), jnp.float32),
            pltpu.SemaphoreType.DMA,
        ],
    )
    def agg(g_hbm, src_hbm, dst_hbm, zeros_hbm, out_hbm,
            acc_sh, src_v, dst_v, rows_v, sem):
        cid = lax.axis_index("c")
        sid = lax.axis_index("s")
        wid = cid * NS + sid
        row0 = sid * rpt

        @pl.when(cid == 0)
        def _():
            pltpu.sync_copy(g_hbm.at[pl.ds(row0, rpt)], acc_sh.at[pl.ds(row0, rpt)])

        @pl.when(cid != 0)
        def _():
            pltpu.sync_copy(zeros_hbm.at[pl.ds(row0, rpt)], acc_sh.at[pl.ds(row0, rpt)])

        plsc.subcore_barrier()
        ebase = wid * epw

        def body(i, carry):
            off = pl.multiple_of(ebase + i * CHUNK, 8)
            pltpu.sync_copy(src_hbm.at[pl.ds(off, CHUNK)], src_v)
            pltpu.sync_copy(dst_hbm.at[pl.ds(off, CHUNK)], dst_v)
            pltpu.async_copy(g_hbm.at[src_v], rows_v, sem).wait()
            pltpu.sync_copy(rows_v, acc_sh.at[dst_v], add=True)
            return carry

        lax.fori_loop(0, nch, body, 0)
        plsc.subcore_barrier()
        pltpu.sync_copy(acc_sh.at[pl.ds(row0, rpt)], out_hbm.at[cid, pl.ds(row0, rpt)])

    return agg


def _dis(d0, d1):
    return lax.rsqrt(d0[...] + d1[...] + 1.0)


def _scale_mm_body(d0, d1, x, w, o):
    dis = _dis(d0, d1)
    o[...] = jnp.dot(x[...], w[...], preferred_element_type=jnp.float32) * dis


def _mid_body(d0, d1, a0, a1, b, w, o):
    dis = _dis(d0, d1)
    h = jax.nn.relu(dis * (a0[...] + a1[...]) + b[...])
    o[...] = jnp.dot(h, w[...], preferred_element_type=jnp.float32) * dis


def _fin_body(d0, d1, a0, a1, b, o):
    dis = _dis(d0, d1)
    o[...] = jax.nn.relu(dis * (a0[...] + a1[...]) + b[...])


def kernel(x, edge_index, W1, b1, W2, b2):
    N, D = x.shape
    E = edge_index.shape[1]
    src = edge_index[0].astype(jnp.int32)
    dst = edge_index[1].astype(jnp.int32)

    ones_w = jnp.ones((CHUNK, WD), jnp.float32)
    zeros_wd = jnp.zeros((N, WD), jnp.float32)
    zeros_nd = jnp.zeros((N, D), jnp.float32)

    degp = _make_deg(N, E)(dst, ones_w, zeros_wd)
    d0 = degp[0, :, 0:1]
    d1 = degp[1, :, 0:1]

    BR = 1000
    grid = (N // BR,)
    col = pl.BlockSpec((BR, 1), lambda i: (i, 0))
    mat = pl.BlockSpec((BR, D), lambda i: (i, 0))
    wspec = pl.BlockSpec((D, D), lambda i: (0, 0))
    bspec = pl.BlockSpec((1, D), lambda i: (0, 0))
    out_nd = jax.ShapeDtypeStruct((N, D), jnp.float32)

    g1 = pl.pallas_call(
        _scale_mm_body, grid=grid,
        in_specs=[col, col, mat, wspec], out_specs=mat, out_shape=out_nd,
    )(d0, d1, x, W1)

    agg = _make_agg(N, D, E)
    acc = agg(g1, src, dst, zeros_nd)

    g2 = pl.pallas_call(
        _mid_body, grid=grid,
        in_specs=[col, col, mat, mat, bspec, wspec], out_specs=mat,
        out_shape=out_nd,
    )(d0, d1, acc[0], acc[1], b1.reshape(1, D), W2)

    acc2 = agg(g2, src, dst, zeros_nd)

    out = pl.pallas_call(
        _fin_body, grid=grid,
        in_specs=[col, col, mat, mat, bspec], out_specs=mat,
        out_shape=out_nd,
    )(d0, d1, acc2[0], acc2[1], b2.reshape(1, D))
    return out


# speedup vs baseline: 9.8699x; 9.8699x over previous
import functools

import jax
import jax.numpy as jnp
from jax import lax
from jax.experimental import pallas as pl
from jax.experimental.pallas import tpu as pltpu
from jax.experimental.pallas import tpu_sc as plsc

NC = 2
NS = 16
CHUNK = 80


def _make_agg(N, D, E):
    NW = NC * NS
    epw = E // NW
    nch = epw // CHUNK
    rpt = N // NS
    mesh = plsc.VectorSubcoreMesh(core_axis_name="c", subcore_axis_name="s", num_cores=NC, num_subcores=NS)

    @functools.partial(
        pl.kernel,
        out_type=jax.ShapeDtypeStruct((NC, N, D), jnp.float32),
        mesh=mesh,
        scratch_types=[
            pltpu.VMEM_SHARED((N, D), jnp.float32),
            pltpu.VMEM((CHUNK,), jnp.int32),
            pltpu.VMEM((CHUNK,), jnp.int32),
            pltpu.VMEM((CHUNK, D), jnp.float32),
            pltpu.SemaphoreType.DMA,
        ],
    )
    def agg(g_hbm, src_hbm, dst_hbm, zeros_hbm, out_hbm,
            acc_sh, src_v, dst_v, rows_v, sem):
        cid = lax.axis_index("c")
        sid = lax.axis_index("s")
        wid = cid * NS + sid
        row0 = sid * rpt

        @pl.when(cid == 0)
        def _():
            pltpu.sync_copy(g_hbm.at[pl.ds(row0, rpt)], acc_sh.at[pl.ds(row0, rpt)])

        @pl.when(cid != 0)
        def _():
            pltpu.sync_copy(zeros_hbm.at[pl.ds(row0, rpt)], acc_sh.at[pl.ds(row0, rpt)])

        plsc.subcore_barrier()
        ebase = wid * epw

        def body(i, carry):
            off = pl.multiple_of(ebase + i * CHUNK, 8)
            pltpu.sync_copy(src_hbm.at[pl.ds(off, CHUNK)], src_v)
            pltpu.sync_copy(dst_hbm.at[pl.ds(off, CHUNK)], dst_v)
            pltpu.async_copy(g_hbm.at[src_v], rows_v, sem).wait()
            pltpu.sync_copy(rows_v, acc_sh.at[dst_v], add=True)
            return carry

        lax.fori_loop(0, nch, body, 0)
        plsc.subcore_barrier()
        pltpu.sync_copy(acc_sh.at[pl.ds(row0, rpt)], out_hbm.at[cid, pl.ds(row0, rpt)])

    return agg


def _dis(d0, d1):
    return lax.rsqrt(d0[...] + d1[...])


def _scale_mm_body(d0, d1, x, w, o):
    dis = _dis(d0, d1)
    o[...] = jnp.dot(x[...], w[...], preferred_element_type=jnp.float32) * dis


def _mid_body(d0, d1, a0, a1, b, w, o):
    dis = _dis(d0, d1)
    h = jax.nn.relu(dis * (a0[...] + a1[...]) + b[...])
    o[...] = jnp.dot(h, w[...], preferred_element_type=jnp.float32) * dis


def _fin_body(d0, d1, a0, a1, b, o):
    dis = _dis(d0, d1)
    o[...] = jax.nn.relu(dis * (a0[...] + a1[...]) + b[...])


def kernel(x, edge_index, W1, b1, W2, b2):
    N, D = x.shape
    E = edge_index.shape[1]
    src = edge_index[0].astype(jnp.int32)
    dst = edge_index[1].astype(jnp.int32)

    NP = ((N + 1279) // 1280) * 1280
    x = jnp.pad(x, ((0, NP - N), (0, 0)))

    ones_nd = jnp.ones((NP, D), jnp.float32)
    zeros_nd = jnp.zeros((NP, D), jnp.float32)

    agg = _make_agg(NP, D, E)

    degp = agg(ones_nd, src, dst, zeros_nd)
    d0 = degp[0][:, 0:1]
    d1 = degp[1][:, 0:1]

    BR = NP // 16
    grid = (NP // BR,)
    col = pl.BlockSpec((BR, 1), lambda i: (i, 0))
    mat = pl.BlockSpec((BR, D), lambda i: (i, 0))
    wspec = pl.BlockSpec((D, D), lambda i: (0, 0))
    bspec = pl.BlockSpec((1, D), lambda i: (0, 0))
    out_nd = jax.ShapeDtypeStruct((NP, D), jnp.float32)

    g1 = pl.pallas_call(
        _scale_mm_body, grid=grid,
        in_specs=[col, col, mat, wspec], out_specs=mat, out_shape=out_nd,
    )(d0, d1, x, W1)

    acc = agg(g1, src, dst, zeros_nd)

    g2 = pl.pallas_call(
        _mid_body, grid=grid,
        in_specs=[col, col, mat, mat, bspec, wspec], out_specs=mat,
        out_shape=out_nd,
    )(d0, d1, acc[0], acc[1], b1.reshape(1, D), W2)

    acc2 = agg(g2, src, dst, zeros_nd)

    out = pl.pallas_call(
        _fin_body, grid=grid,
        in_specs=[col, col, mat, mat, bspec], out_specs=mat,
        out_shape=out_nd,
    )(d0, d1, acc2[0], acc2[1], b2.reshape(1, D))
    return out[:N]


# speedup vs baseline: 25.2455x; 2.5578x over previous
import functools

import jax
import jax.numpy as jnp
from jax import lax
from jax.experimental import pallas as pl
from jax.experimental.pallas import tpu as pltpu
from jax.experimental.pallas import tpu_sc as plsc

NC = 2
NS = 16
CHUNK = 80


IRING = 8
RRING = 4
A_IDX = 4
A_GAT = 2


def _make_agg(N, D, E, gather=True):
    NW = NC * NS
    epw = E // NW
    nch = epw // CHUNK
    rpt = N // NS
    assert nch > IRING
    mesh = plsc.VectorSubcoreMesh(core_axis_name="c", subcore_axis_name="s",
                                  num_cores=NC, num_subcores=NS)
    nrows = RRING * CHUNK if gather else CHUNK

    @functools.partial(
        pl.kernel,
        out_type=jax.ShapeDtypeStruct((NC, N, D), jnp.float32),
        mesh=mesh,
        scratch_types=[
            pltpu.VMEM_SHARED((N, D), jnp.float32),
            pltpu.VMEM((IRING, CHUNK), jnp.int32),
            pltpu.VMEM((IRING, CHUNK), jnp.int32),
            pltpu.VMEM((nrows, D), jnp.float32),
            pltpu.SemaphoreType.DMA((IRING,)),
            pltpu.SemaphoreType.DMA((RRING,)),
            pltpu.SemaphoreType.DMA((RRING,)),
        ],
    )
    def agg(g_hbm, src_hbm, dst_hbm, zeros_hbm, out_hbm,
            acc_sh, src_i, dst_i, rows, isem, gsem, ssem):
        cid = lax.axis_index("c")
        sid = lax.axis_index("s")
        wid = cid * NS + sid
        row0 = sid * rpt

        @pl.when(cid == 0)
        def _():
            pltpu.sync_copy(g_hbm.at[pl.ds(row0, rpt)], acc_sh.at[pl.ds(row0, rpt)])

        @pl.when(cid != 0)
        def _():
            pltpu.sync_copy(zeros_hbm.at[pl.ds(row0, rpt)], acc_sh.at[pl.ds(row0, rpt)])

        ebase = wid * epw

        def echunk(j):
            return pl.ds(pl.multiple_of(ebase + j * CHUNK, 8), CHUNK)

        def idx_start(j):
            s = lax.rem(j, IRING)
            if gather:
                pltpu.async_copy(src_hbm.at[echunk(j)], src_i.at[s], isem.at[s])
            pltpu.async_copy(dst_hbm.at[echunk(j)], dst_i.at[s], isem.at[s])

        def idx_wait(j):
            s = lax.rem(j, IRING)
            if gather:
                pltpu.make_async_copy(
                    src_hbm.at[echunk(j)], src_i.at[s], isem.at[s]).wait()
            pltpu.make_async_copy(
                dst_hbm.at[echunk(j)], dst_i.at[s], isem.at[s]).wait()

        def rows_at(j):
            if not gather:
                return rows
            return rows.at[pl.ds(lax.rem(j, RRING) * CHUNK, CHUNK)]

        def gather_start(j):
            s = lax.rem(j, IRING)
            pltpu.async_copy(g_hbm.at[src_i.at[s]], rows_at(j),
                             gsem.at[lax.rem(j, RRING)])

        def gather_wait(j):
            s = lax.rem(j, IRING)
            pltpu.make_async_copy(g_hbm.at[src_i.at[s]], rows_at(j),
                                  gsem.at[lax.rem(j, RRING)]).wait()

        def scatter_start(j):
            pltpu.async_copy(rows_at(j), acc_sh.at[dst_i.at[lax.rem(j, IRING)]],
                             ssem.at[lax.rem(j, RRING)], add=True)

        def scatter_wait(j):
            pltpu.make_async_copy(rows_at(j),
                                  acc_sh.at[dst_i.at[lax.rem(j, IRING)]],
                                  ssem.at[lax.rem(j, RRING)]).wait()

        if not gather:
            pltpu.sync_copy(g_hbm.at[pl.ds(0, CHUNK)], rows)
        plsc.subcore_barrier()

        for k in range(A_IDX):
            idx_start(k)
        if gather:
            for k in range(A_GAT):
                idx_wait(k)
                gather_start(k)

        def body(i, carry):
            if gather:
                gather_wait(i)
            else:
                idx_wait(i)

                @pl.when(i >= RRING)
                def _():
                    scatter_wait(i - RRING)

            scatter_start(i)
            j2 = i + A_IDX

            @pl.when(j2 < nch)
            def _():
                idx_start(j2)

            if gather:
                j = i + A_GAT

                @pl.when(j < nch)
                def _():
                    @pl.when(j >= RRING)
                    def _():
                        scatter_wait(j - RRING)
                    idx_wait(j)
                    gather_start(j)
            return carry

        lax.fori_loop(0, nch, body, 0)
        for t in range(RRING):
            scatter_wait(nch - RRING + t)
        plsc.subcore_barrier()
        pltpu.sync_copy(acc_sh.at[pl.ds(row0, rpt)], out_hbm.at[cid, pl.ds(row0, rpt)])

    return agg


def _dis(d0, d1):
    return lax.rsqrt(d0[...] + d1[...])


def _scale_mm_body(d0, d1, x, w, o):
    dis = _dis(d0, d1)
    o[...] = jnp.dot(x[...], w[...], preferred_element_type=jnp.float32) * dis


def _mid_body(d0, d1, a0, a1, b, w, o):
    dis = _dis(d0, d1)
    h = jax.nn.relu(dis * (a0[...] + a1[...]) + b[...])
    o[...] = jnp.dot(h, w[...], preferred_element_type=jnp.float32) * dis


def _fin_body(d0, d1, a0, a1, b, o):
    dis = _dis(d0, d1)
    o[...] = jax.nn.relu(dis * (a0[...] + a1[...]) + b[...])


def kernel(x, edge_index, W1, b1, W2, b2):
    N, D = x.shape
    E = edge_index.shape[1]
    src = edge_index[0].astype(jnp.int32)
    dst = edge_index[1].astype(jnp.int32)

    NP = ((N + 127) // 128) * 128
    x = jnp.pad(x, ((0, NP - N), (0, 0)))

    ones_nd = jnp.ones((NP, D), jnp.float32)
    zeros_nd = jnp.zeros((NP, D), jnp.float32)

    agg = _make_agg(NP, D, E)

    degp = _make_agg(NP, D, E, gather=False)(ones_nd, src, dst, zeros_nd)
    d0 = degp[0][:, 0:1]
    d1 = degp[1][:, 0:1]

    BR = NP // 16
    grid = (NP // BR,)
    col = pl.BlockSpec((BR, 1), lambda i: (i, 0))
    mat = pl.BlockSpec((BR, D), lambda i: (i, 0))
    wspec = pl.BlockSpec((D, D), lambda i: (0, 0))
    bspec = pl.BlockSpec((1, D), lambda i: (0, 0))
    out_nd = jax.ShapeDtypeStruct((NP, D), jnp.float32)

    g1 = pl.pallas_call(
        _scale_mm_body, grid=grid,
        in_specs=[col, col, mat, wspec], out_specs=mat, out_shape=out_nd,
    )(d0, d1, x, W1)

    acc = agg(g1, src, dst, zeros_nd)

    g2 = pl.pallas_call(
        _mid_body, grid=grid,
        in_specs=[col, col, mat, mat, bspec, wspec], out_specs=mat,
        out_shape=out_nd,
    )(d0, d1, acc[0], acc[1], b1.reshape(1, D), W2)

    acc2 = agg(g2, src, dst, zeros_nd)

    out = pl.pallas_call(
        _fin_body, grid=grid,
        in_specs=[col, col, mat, mat, bspec], out_specs=mat,
        out_shape=out_nd,
    )(d0, d1, acc2[0], acc2[1], b2.reshape(1, D))
    return out[:N]


# speedup vs baseline: 28.0914x; 1.1127x over previous
import functools

import jax
import jax.numpy as jnp
from jax import lax
from jax.experimental import pallas as pl
from jax.experimental.pallas import tpu as pltpu
from jax.experimental.pallas import tpu_sc as plsc

NC = 2
NS = 16
CHUNK = 80


IRING = 8
RRING = 4
A_IDX = 4
A_GAT = 2


def _make_agg(N, D, E, gather=True):
    NW = NC * NS
    epw = E // NW
    nch = epw // CHUNK
    rpt = N // NS
    assert nch > IRING
    mesh = plsc.VectorSubcoreMesh(core_axis_name="c", subcore_axis_name="s",
                                  num_cores=NC, num_subcores=NS)
    nrows = RRING * CHUNK if gather else CHUNK
    wd = D if gather else 16

    cparams = None if gather else pltpu.CompilerParams(use_tc_tiling_on_sc=False)

    @functools.partial(
        pl.kernel,
        out_type=jax.ShapeDtypeStruct((NC, N, wd), jnp.float32),
        mesh=mesh,
        compiler_params=cparams,
        scratch_types=[
            pltpu.VMEM_SHARED((N, wd), jnp.float32),
            pltpu.VMEM((IRING, CHUNK), jnp.int32),
            pltpu.VMEM((IRING, CHUNK), jnp.int32),
            pltpu.VMEM((nrows, wd), jnp.float32),
            pltpu.SemaphoreType.DMA((IRING,)),
            pltpu.SemaphoreType.DMA((RRING,)),
            pltpu.SemaphoreType.DMA((RRING,)),
        ],
    )
    def agg(g_hbm, src_hbm, dst_hbm, zeros_hbm, out_hbm,
            acc_sh, src_i, dst_i, rows, isem, gsem, ssem):
        cid = lax.axis_index("c")
        sid = lax.axis_index("s")
        wid = cid * NS + sid
        row0 = sid * rpt

        @pl.when(cid == 0)
        def _():
            pltpu.sync_copy(g_hbm.at[pl.ds(row0, rpt)], acc_sh.at[pl.ds(row0, rpt)])

        @pl.when(cid != 0)
        def _():
            pltpu.sync_copy(zeros_hbm.at[pl.ds(row0, rpt)], acc_sh.at[pl.ds(row0, rpt)])

        ebase = wid * epw

        def echunk(j):
            return pl.ds(pl.multiple_of(ebase + j * CHUNK, 8), CHUNK)

        def idx_start(j):
            s = lax.rem(j, IRING)
            if gather:
                pltpu.async_copy(src_hbm.at[echunk(j)], src_i.at[s], isem.at[s])
            pltpu.async_copy(dst_hbm.at[echunk(j)], dst_i.at[s], isem.at[s])

        def idx_wait(j):
            s = lax.rem(j, IRING)
            if gather:
                pltpu.make_async_copy(
                    src_hbm.at[echunk(j)], src_i.at[s], isem.at[s]).wait()
            pltpu.make_async_copy(
                dst_hbm.at[echunk(j)], dst_i.at[s], isem.at[s]).wait()

        def rows_at(j):
            if not gather:
                return rows
            return rows.at[pl.ds(lax.rem(j, RRING) * CHUNK, CHUNK)]

        def gather_start(j):
            s = lax.rem(j, IRING)
            pltpu.async_copy(g_hbm.at[src_i.at[s]], rows_at(j),
                             gsem.at[lax.rem(j, RRING)])

        def gather_wait(j):
            s = lax.rem(j, IRING)
            pltpu.make_async_copy(g_hbm.at[src_i.at[s]], rows_at(j),
                                  gsem.at[lax.rem(j, RRING)]).wait()

        def scatter_start(j):
            pltpu.async_copy(rows_at(j), acc_sh.at[dst_i.at[lax.rem(j, IRING)]],
                             ssem.at[lax.rem(j, RRING)], add=True)

        def scatter_wait(j):
            pltpu.make_async_copy(rows_at(j),
                                  acc_sh.at[dst_i.at[lax.rem(j, IRING)]],
                                  ssem.at[lax.rem(j, RRING)]).wait()

        if not gather:
            pltpu.sync_copy(g_hbm.at[pl.ds(0, CHUNK)], rows)
        plsc.subcore_barrier()

        for k in range(A_IDX):
            idx_start(k)
        if gather:
            for k in range(A_GAT):
                idx_wait(k)
                gather_start(k)

        def body(i, carry):
            if gather:
                gather_wait(i)
            else:
                idx_wait(i)

                @pl.when(i >= RRING)
                def _():
                    scatter_wait(i - RRING)

            scatter_start(i)
            j2 = i + A_IDX

            @pl.when(j2 < nch)
            def _():
                idx_start(j2)

            if gather:
                j = i + A_GAT

                @pl.when(j < nch)
                def _():
                    @pl.when(j >= RRING)
                    def _():
                        scatter_wait(j - RRING)
                    idx_wait(j)
                    gather_start(j)
            return carry

        lax.fori_loop(0, nch, body, 0)
        for t in range(RRING):
            scatter_wait(nch - RRING + t)
        plsc.subcore_barrier()
        pltpu.sync_copy(acc_sh.at[pl.ds(row0, rpt)], out_hbm.at[cid, pl.ds(row0, rpt)])

    return agg


def _dis(d0, d1):
    return lax.rsqrt(d0[...] + d1[...])


def _scale_mm_body(d0, d1, x, w, o):
    dis = _dis(d0, d1)
    o[...] = jnp.dot(x[...], w[...], preferred_element_type=jnp.float32) * dis


def _mid_body(d0, d1, a0, a1, b, w, o):
    dis = _dis(d0, d1)
    h = jax.nn.relu(dis * (a0[...] + a1[...]) + b[...])
    o[...] = jnp.dot(h, w[...], preferred_element_type=jnp.float32) * dis


def _fin_body(d0, d1, a0, a1, b, o):
    dis = _dis(d0, d1)
    o[...] = jax.nn.relu(dis * (a0[...] + a1[...]) + b[...])


def kernel(x, edge_index, W1, b1, W2, b2):
    N, D = x.shape
    E = edge_index.shape[1]
    src = edge_index[0].astype(jnp.int32)
    dst = edge_index[1].astype(jnp.int32)

    NP = ((N + 127) // 128) * 128
    x = jnp.pad(x, ((0, NP - N), (0, 0)))

    zeros_nd = jnp.zeros((NP, D), jnp.float32)

    agg = _make_agg(NP, D, E)

    degp = _make_agg(NP, D, E, gather=False)(
        jnp.ones((NP, 16), jnp.float32), src, dst,
        jnp.zeros((NP, 16), jnp.float32))
    d0 = degp[0][:, 0:1]
    d1 = degp[1][:, 0:1]

    BR = NP // 16
    grid = (NP // BR,)
    col = pl.BlockSpec((BR, 1), lambda i: (i, 0))
    mat = pl.BlockSpec((BR, D), lambda i: (i, 0))
    wspec = pl.BlockSpec((D, D), lambda i: (0, 0))
    bspec = pl.BlockSpec((1, D), lambda i: (0, 0))
    out_nd = jax.ShapeDtypeStruct((NP, D), jnp.float32)

    g1 = pl.pallas_call(
        _scale_mm_body, grid=grid,
        in_specs=[col, col, mat, wspec], out_specs=mat, out_shape=out_nd,
    )(d0, d1, x, W1)

    acc = agg(g1, src, dst, zeros_nd)

    g2 = pl.pallas_call(
        _mid_body, grid=grid,
        in_specs=[col, col, mat, mat, bspec, wspec], out_specs=mat,
        out_shape=out_nd,
    )(d0, d1, acc[0], acc[1], b1.reshape(1, D), W2)

    acc2 = agg(g2, src, dst, zeros_nd)

    out = pl.pallas_call(
        _fin_body, grid=grid,
        in_specs=[col, col, mat, mat, bspec], out_specs=mat,
        out_shape=out_nd,
    )(d0, d1, acc2[0], acc2[1], b2.reshape(1, D))
    return out[:N]


# speedup vs baseline: 30.1065x; 1.0717x over previous
import functools

import jax
import jax.numpy as jnp
from jax import lax
from jax.experimental import pallas as pl
from jax.experimental.pallas import tpu as pltpu
from jax.experimental.pallas import tpu_sc as plsc

NC = 2
NS = 16
CHUNK = 80


IRING = 8
RRING = 4
A_IDX = 5
A_GAT = 2


def _make_agg(N, D, E, gather=True):
    NW = NC * NS
    epw = E // NW
    nch = epw // CHUNK
    rpt = N // NS
    assert nch > IRING
    mesh = plsc.VectorSubcoreMesh(core_axis_name="c", subcore_axis_name="s",
                                  num_cores=NC, num_subcores=NS)
    nrows = RRING * CHUNK if gather else CHUNK
    wd = D if gather else 16

    cparams = None if gather else pltpu.CompilerParams(use_tc_tiling_on_sc=False)

    @functools.partial(
        pl.kernel,
        out_type=jax.ShapeDtypeStruct((NC, N, wd), jnp.float32),
        mesh=mesh,
        compiler_params=cparams,
        scratch_types=[
            pltpu.VMEM_SHARED((N, wd), jnp.float32),
            pltpu.VMEM((IRING, CHUNK), jnp.int32),
            pltpu.VMEM((IRING, CHUNK), jnp.int32),
            pltpu.VMEM((nrows, wd), jnp.float32),
            pltpu.SemaphoreType.DMA((IRING,)),
            pltpu.SemaphoreType.DMA((RRING,)),
            pltpu.SemaphoreType.DMA((RRING,)),
        ],
    )
    def agg(g_hbm, src_hbm, dst_hbm, zeros_hbm, out_hbm,
            acc_sh, src_i, dst_i, rows, isem, gsem, ssem):
        cid = lax.axis_index("c")
        sid = lax.axis_index("s")
        wid = cid * NS + sid
        row0 = sid * rpt

        @pl.when(cid == 0)
        def _():
            pltpu.sync_copy(g_hbm.at[pl.ds(row0, rpt)], acc_sh.at[pl.ds(row0, rpt)])

        @pl.when(cid != 0)
        def _():
            pltpu.sync_copy(zeros_hbm.at[pl.ds(row0, rpt)], acc_sh.at[pl.ds(row0, rpt)])

        ebase = wid * epw

        def echunk(j):
            return pl.ds(pl.multiple_of(ebase + j * CHUNK, 8), CHUNK)

        def idx_start(j):
            s = lax.rem(j, IRING)
            if gather:
                pltpu.async_copy(src_hbm.at[echunk(j)], src_i.at[s], isem.at[s])
            pltpu.async_copy(dst_hbm.at[echunk(j)], dst_i.at[s], isem.at[s])

        def idx_wait(j):
            s = lax.rem(j, IRING)
            if gather:
                pltpu.make_async_copy(
                    src_hbm.at[echunk(j)], src_i.at[s], isem.at[s]).wait()
            pltpu.make_async_copy(
                dst_hbm.at[echunk(j)], dst_i.at[s], isem.at[s]).wait()

        def rows_at(j):
            if not gather:
                return rows
            return rows.at[pl.ds(lax.rem(j, RRING) * CHUNK, CHUNK)]

        def gather_start(j):
            s = lax.rem(j, IRING)
            pltpu.async_copy(g_hbm.at[src_i.at[s]], rows_at(j),
                             gsem.at[lax.rem(j, RRING)])

        def gather_wait(j):
            s = lax.rem(j, IRING)
            pltpu.make_async_copy(g_hbm.at[src_i.at[s]], rows_at(j),
                                  gsem.at[lax.rem(j, RRING)]).wait()

        def scatter_start(j):
            pltpu.async_copy(rows_at(j), acc_sh.at[dst_i.at[lax.rem(j, IRING)]],
                             ssem.at[lax.rem(j, RRING)], add=True)

        def scatter_wait(j):
            pltpu.make_async_copy(rows_at(j),
                                  acc_sh.at[dst_i.at[lax.rem(j, IRING)]],
                                  ssem.at[lax.rem(j, RRING)]).wait()

        if not gather:
            pltpu.sync_copy(g_hbm.at[pl.ds(0, CHUNK)], rows)
        plsc.subcore_barrier()

        for k in range(A_IDX):
            idx_start(k)
        if gather:
            for k in range(A_GAT):
                idx_wait(k)
                gather_start(k)

        def body(i, carry):
            if gather:
                gather_wait(i)
            else:
                idx_wait(i)

                @pl.when(i >= RRING)
                def _():
                    scatter_wait(i - RRING)

            scatter_start(i)
            j2 = i + A_IDX

            @pl.when(j2 < nch)
            def _():
                idx_start(j2)

            if gather:
                j = i + A_GAT

                @pl.when(j < nch)
                def _():
                    @pl.when(j >= RRING)
                    def _():
                        scatter_wait(j - RRING)
                    idx_wait(j)
                    gather_start(j)
            return carry

        lax.fori_loop(0, nch, body, 0)
        for t in range(RRING):
            scatter_wait(nch - RRING + t)
        plsc.subcore_barrier()
        pltpu.sync_copy(acc_sh.at[pl.ds(row0, rpt)], out_hbm.at[cid, pl.ds(row0, rpt)])

    return agg


def _dis(dref):
    return lax.rsqrt(dref[0] + dref[1])


def _scale_mm_body(dref, x, w, o):
    dis = _dis(dref)
    o[...] = jnp.dot(x[...], w[...], preferred_element_type=jnp.float32) * dis


def _mid_body(dref, aref, b, w, o):
    dis = _dis(dref)
    h = jax.nn.relu(dis * (aref[0] + aref[1]) + b[...])
    o[...] = jnp.dot(h, w[...], preferred_element_type=jnp.float32) * dis


def _fin_body(dref, aref, b, o):
    dis = _dis(dref)
    o[...] = jax.nn.relu(dis * (aref[0] + aref[1]) + b[...])


def kernel(x, edge_index, W1, b1, W2, b2):
    N, D = x.shape
    E = edge_index.shape[1]
    src = edge_index[0].astype(jnp.int32)
    dst = edge_index[1].astype(jnp.int32)

    NP = ((N + 127) // 128) * 128
    x = jnp.pad(x, ((0, NP - N), (0, 0)))

    zeros_nd = jnp.zeros((NP, D), jnp.float32)

    agg = _make_agg(NP, D, E)

    degp = _make_agg(NP, D, E, gather=False)(
        jnp.ones((NP, 16), jnp.float32), src, dst,
        jnp.zeros((NP, 16), jnp.float32))

    BR = NP // 16
    grid = (NP // BR,)
    dspec = pl.BlockSpec((2, BR, 1), lambda i: (0, i, 0))
    aspec = pl.BlockSpec((2, BR, D), lambda i: (0, i, 0))
    mat = pl.BlockSpec((BR, D), lambda i: (i, 0))
    wspec = pl.BlockSpec((D, D), lambda i: (0, 0))
    bspec = pl.BlockSpec((1, D), lambda i: (0, 0))
    out_nd = jax.ShapeDtypeStruct((NP, D), jnp.float32)

    g1 = pl.pallas_call(
        _scale_mm_body, grid=grid,
        in_specs=[dspec, mat, wspec], out_specs=mat, out_shape=out_nd,
    )(degp[:, :, 0:1], x, W1)

    acc = agg(g1, src, dst, zeros_nd)

    g2 = pl.pallas_call(
        _mid_body, grid=grid,
        in_specs=[dspec, aspec, bspec, wspec], out_specs=mat,
        out_shape=out_nd,
    )(degp[:, :, 0:1], acc, b1.reshape(1, D), W2)

    acc2 = agg(g2, src, dst, zeros_nd)

    out = pl.pallas_call(
        _fin_body, grid=grid,
        in_specs=[dspec, aspec, bspec], out_specs=mat,
        out_shape=out_nd,
    )(degp[:, :, 0:1], acc2, b2.reshape(1, D))
    return out[:N]


# speedup vs baseline: 30.6010x; 1.0164x over previous
import functools

import jax
import jax.numpy as jnp
from jax import lax
from jax.experimental import pallas as pl
from jax.experimental.pallas import tpu as pltpu
from jax.experimental.pallas import tpu_sc as plsc

NC = 2
NS = 16
CHUNK = 80


IRING = 8
RRING = 4
A_IDX = 5
A_GAT = 2


def _make_agg(N, D, E, gather=True):
    NW = NC * NS
    epw = E // NW
    nch = epw // CHUNK
    rpt = N // NS
    assert nch > IRING
    mesh = plsc.VectorSubcoreMesh(core_axis_name="c", subcore_axis_name="s",
                                  num_cores=NC, num_subcores=NS)
    nrows = RRING * CHUNK if gather else CHUNK
    wd = D if gather else 16

    cparams = None if gather else pltpu.CompilerParams(use_tc_tiling_on_sc=False)

    @functools.partial(
        pl.kernel,
        out_type=jax.ShapeDtypeStruct((NC, N, wd), jnp.float32),
        mesh=mesh,
        compiler_params=cparams,
        scratch_types=[
            pltpu.VMEM_SHARED((N, wd), jnp.float32),
            pltpu.VMEM((IRING, CHUNK), jnp.int32),
            pltpu.VMEM((IRING, CHUNK), jnp.int32),
            pltpu.VMEM((nrows, wd), jnp.float32),
            pltpu.SemaphoreType.DMA((IRING,)),
            pltpu.SemaphoreType.DMA((RRING,)),
            pltpu.SemaphoreType.DMA((RRING,)),
        ],
    )
    def agg(g_hbm, src_hbm, dst_hbm, zeros_hbm, out_hbm,
            acc_sh, src_i, dst_i, rows, isem, gsem, ssem):
        cid = lax.axis_index("c")
        sid = lax.axis_index("s")
        wid = cid * NS + sid
        row0 = sid * rpt

        @pl.when(cid == 0)
        def _():
            pltpu.sync_copy(g_hbm.at[pl.ds(row0, rpt)], acc_sh.at[pl.ds(row0, rpt)])

        @pl.when(cid != 0)
        def _():
            pltpu.sync_copy(zeros_hbm, acc_sh.at[pl.ds(row0, rpt)])

        ebase = wid * epw

        def echunk(j):
            return pl.ds(pl.multiple_of(ebase + j * CHUNK, 8), CHUNK)

        def idx_start(j):
            s = lax.rem(j, IRING)
            if gather:
                pltpu.async_copy(src_hbm.at[echunk(j)], src_i.at[s], isem.at[s])
            pltpu.async_copy(dst_hbm.at[echunk(j)], dst_i.at[s], isem.at[s])

        def idx_wait(j):
            s = lax.rem(j, IRING)
            if gather:
                pltpu.make_async_copy(
                    src_hbm.at[echunk(j)], src_i.at[s], isem.at[s]).wait()
            pltpu.make_async_copy(
                dst_hbm.at[echunk(j)], dst_i.at[s], isem.at[s]).wait()

        def rows_at(j):
            if not gather:
                return rows
            return rows.at[pl.ds(lax.rem(j, RRING) * CHUNK, CHUNK)]

        def gather_start(j):
            s = lax.rem(j, IRING)
            pltpu.async_copy(g_hbm.at[src_i.at[s]], rows_at(j),
                             gsem.at[lax.rem(j, RRING)])

        def gather_wait(j):
            s = lax.rem(j, IRING)
            pltpu.make_async_copy(g_hbm.at[src_i.at[s]], rows_at(j),
                                  gsem.at[lax.rem(j, RRING)]).wait()

        def scatter_start(j):
            pltpu.async_copy(rows_at(j), acc_sh.at[dst_i.at[lax.rem(j, IRING)]],
                             ssem.at[lax.rem(j, RRING)], add=True)

        def scatter_wait(j):
            pltpu.make_async_copy(rows_at(j),
                                  acc_sh.at[dst_i.at[lax.rem(j, IRING)]],
                                  ssem.at[lax.rem(j, RRING)]).wait()

        if not gather:
            pltpu.sync_copy(g_hbm.at[pl.ds(0, CHUNK)], rows)
        plsc.subcore_barrier()

        for k in range(A_IDX):
            idx_start(k)
        if gather:
            for k in range(A_GAT):
                idx_wait(k)
                gather_start(k)

        def body(i, carry):
            if gather:
                gather_wait(i)
            else:
                idx_wait(i)

                @pl.when(i >= RRING)
                def _():
                    scatter_wait(i - RRING)

            scatter_start(i)
            j2 = i + A_IDX

            @pl.when(j2 < nch)
            def _():
                idx_start(j2)

            if gather:
                j = i + A_GAT

                @pl.when(j < nch)
                def _():
                    @pl.when(j >= RRING)
                    def _():
                        scatter_wait(j - RRING)
                    idx_wait(j)
                    gather_start(j)
            return carry

        lax.fori_loop(0, nch, body, 0)
        for t in range(RRING):
            scatter_wait(nch - RRING + t)
        plsc.subcore_barrier()
        pltpu.sync_copy(acc_sh.at[pl.ds(row0, rpt)], out_hbm.at[cid, pl.ds(row0, rpt)])

    return agg


def _dis(dref):
    return lax.rsqrt(dref[0][:, 0:1] + dref[1][:, 0:1])


def _scale_mm_body(dref, x, w, o):
    dis = _dis(dref)
    o[...] = jnp.dot(x[...], w[...], preferred_element_type=jnp.float32) * dis


def _mid_body(dref, aref, b, w, o):
    dis = _dis(dref)
    h = jax.nn.relu(dis * (aref[0] + aref[1]) + b[...])
    o[...] = jnp.dot(h, w[...], preferred_element_type=jnp.float32) * dis


def _fin_body(dref, aref, b, o):
    dis = _dis(dref)
    o[...] = jax.nn.relu(dis * (aref[0] + aref[1]) + b[...])


def kernel(x, edge_index, W1, b1, W2, b2):
    N, D = x.shape
    E = edge_index.shape[1]
    src = edge_index[0].astype(jnp.int32)
    dst = edge_index[1].astype(jnp.int32)

    NP = ((N + 127) // 128) * 128
    x = jnp.pad(x, ((0, NP - N), (0, 0)))

    zeros_nd = jnp.zeros((NP // NS, D), jnp.float32)

    agg = _make_agg(NP, D, E)

    degp = _make_agg(NP, D, E, gather=False)(
        jnp.ones((NP, 16), jnp.float32), src, dst,
        jnp.zeros((NP // NS, 16), jnp.float32))

    BR = NP // 16
    grid = (NP // BR,)
    dspec = pl.BlockSpec((2, BR, 16), lambda i: (0, i, 0))
    aspec = pl.BlockSpec((2, BR, D), lambda i: (0, i, 0))
    mat = pl.BlockSpec((BR, D), lambda i: (i, 0))
    wspec = pl.BlockSpec((D, D), lambda i: (0, 0))
    bspec = pl.BlockSpec((1, D), lambda i: (0, 0))
    out_nd = jax.ShapeDtypeStruct((NP, D), jnp.float32)

    g1 = pl.pallas_call(
        _scale_mm_body, grid=grid,
        in_specs=[dspec, mat, wspec], out_specs=mat, out_shape=out_nd,
    )(degp, x, W1)

    acc = agg(g1, src, dst, zeros_nd)

    g2 = pl.pallas_call(
        _mid_body, grid=grid,
        in_specs=[dspec, aspec, bspec, wspec], out_specs=mat,
        out_shape=out_nd,
    )(degp, acc, b1.reshape(1, D), W2)

    acc2 = agg(g2, src, dst, zeros_nd)

    out = pl.pallas_call(
        _fin_body, grid=grid,
        in_specs=[dspec, aspec, bspec], out_specs=mat,
        out_shape=jax.ShapeDtypeStruct((N, D), jnp.float32),
    )(degp, acc2, b2.reshape(1, D))
    return out
